# Initial kernel scaffold; baseline (speedup 1.0000x reference)
import jax
import jax.numpy as jnp
from jax.experimental import pallas as pl


def kernel(x, edge_index, W1, b1, W2, b2, Wfc, bfc):
    raise NotImplementedError("write your pallas kernel here")


# speedup vs baseline: 7.7974x; 7.7974x over previous
import jax
import jax.numpy as jnp
from jax import lax
from jax.experimental import pallas as pl
from jax.experimental.pallas import tpu as pltpu
from jax.experimental.pallas import tpu_sc as plsc

N = 10000
E = 320000
D = 128
NC = 2
NS = 16
NW = NC * NS
CHUNK = 128
NCHUNKS = E // CHUNK
CPW = 79
EPAD = NW * CPW * CHUNK
R = 10112
RPS = R // NS
DEGW = 16

_MESH = dict(core_axis_name="c", subcore_axis_name="s",
             num_cores=NC, num_subcores=NS)
_NOLAYOUT = pltpu.CompilerParams(needs_layout_passes=False)


def _make_sc_scatter():
    def body(y_hbm, src_hbm, dst_hbm, z_hbm, acc_out,
             src_v, dst_v, gb, acc_sh, sem):
        c = lax.axis_index("c")
        s = lax.axis_index("s")
        w = s * NC + c
        base = s * RPS

        pltpu.sync_copy(z_hbm, acc_sh.at[pl.ds(base, RPS)])
        plsc.subcore_barrier()

        pltpu.sync_copy(src_hbm.at[w], src_v)
        pltpu.sync_copy(dst_hbm.at[w], dst_v)
        nreal = jnp.clip(NCHUNKS - w * CPW, 0, CPW)

        def _chunk(j, _):
            pltpu.async_copy(y_hbm.at[src_v.at[j]], gb, sem).wait()
            pltpu.sync_copy(gb, acc_sh.at[dst_v.at[j]], add=True)
            return 0
        lax.fori_loop(0, nreal, _chunk, 0)
        plsc.subcore_barrier()

        pltpu.sync_copy(acc_sh.at[pl.ds(base, RPS)],
                        acc_out.at[c, pl.ds(base, RPS)])

    return pl.kernel(
        body,
        out_type=[jax.ShapeDtypeStruct((NC, R, D), jnp.float32)],
        mesh=plsc.VectorSubcoreMesh(**_MESH),
        scratch_types=[
            pltpu.VMEM((CPW, CHUNK), jnp.int32),
            pltpu.VMEM((CPW, CHUNK), jnp.int32),
            pltpu.VMEM((CHUNK, D), jnp.float32),
            pltpu.VMEM_SHARED((R, D), jnp.float32),
            pltpu.SemaphoreType.DMA,
        ],
        compiler_params=_NOLAYOUT,
        name="sage_sc_scatter")


def _make_sc_deg():
    def body(dst_hbm, z_hbm, deg_out, dst_v, ones_v, deg_sh):
        c = lax.axis_index("c")
        s = lax.axis_index("s")
        w = s * NC + c
        base = s * RPS

        def _fill(i, _):
            for k in range(D // 16):
                ones_v[i, pl.ds(k * 16, 16)] = jnp.ones((16,), jnp.float32)
            return 0
        lax.fori_loop(0, CHUNK, _fill, 0)
        pltpu.sync_copy(z_hbm, deg_sh.at[pl.ds(base, RPS)])
        plsc.subcore_barrier()

        pltpu.sync_copy(dst_hbm.at[w], dst_v)
        nreal = jnp.clip(NCHUNKS - w * CPW, 0, CPW)

        def _chunk(j, _):
            pltpu.sync_copy(ones_v, deg_sh.at[dst_v.at[j]], add=True)
            return 0
        lax.fori_loop(0, nreal, _chunk, 0)
        plsc.subcore_barrier()

        pltpu.sync_copy(deg_sh.at[pl.ds(base, RPS)],
                        deg_out.at[c, pl.ds(base, RPS)])

    return pl.kernel(
        body,
        out_type=[jax.ShapeDtypeStruct((NC, R, D), jnp.float32)],
        mesh=plsc.VectorSubcoreMesh(**_MESH),
        scratch_types=[
            pltpu.VMEM((CPW, CHUNK), jnp.int32),
            pltpu.VMEM((CHUNK, D), jnp.float32),
            pltpu.VMEM_SHARED((R, D), jnp.float32),
        ],
        compiler_params=_NOLAYOUT,
        name="sage_sc_deg")


_sc_scatter = _make_sc_scatter()
_sc_deg = _make_sc_deg()


def _mm_body(x_ref, w_ref, o_ref):
    o_ref[...] = jnp.dot(x_ref[...], w_ref[...],
                         preferred_element_type=jnp.float32,
                         precision=lax.Precision.HIGHEST)


def _tc_matmul(x, w):
    return pl.pallas_call(
        _mm_body,
        out_shape=jax.ShapeDtypeStruct((x.shape[0], w.shape[1]), jnp.float32),
    )(x, w)


def _mid_body(p_ref, y_ref, dp_ref, b_ref, w_ref, y2_ref, inv_ref):
    dp = dp_ref[...]
    deg = dp[0] + dp[1]
    inv = 1.0 / (deg + 1.0)
    p = p_ref[...]
    h1 = (p[0, :N] + p[1, :N] + y_ref[...]) * inv + b_ref[...]
    h1 = jnp.maximum(h1, 0.0)
    y2_ref[...] = jnp.dot(h1, w_ref[...], preferred_element_type=jnp.float32,
                          precision=lax.Precision.HIGHEST)
    inv_ref[...] = inv


def _tc_mid(p, y1, dp, b1, W2):
    return pl.pallas_call(
        _mid_body,
        out_shape=[jax.ShapeDtypeStruct((N, D), jnp.float32),
                   jax.ShapeDtypeStruct((N, 1), jnp.float32)],
    )(p, y1, dp, b1, W2)


def _final_body(q_ref, y2_ref, inv_ref, b_ref, o_ref):
    q = q_ref[...]
    o_ref[...] = ((q[0, :N] + q[1, :N] + y2_ref[...]) * inv_ref[...]
                  + b_ref[...])


def _tc_final(q, y2, inv, b2):
    return pl.pallas_call(
        _final_body,
        out_shape=jax.ShapeDtypeStruct((N, D), jnp.float32),
    )(q, y2, inv, b2)


@jax.jit
def kernel(x, edge_index, W1, b1, W2, b2, Wfc, bfc):
    del Wfc, bfc
    src = edge_index[0]
    dst = edge_index[1]
    npad = EPAD - E
    src_p = jnp.concatenate([src, jnp.zeros((npad,), jnp.int32)])
    dst_p = jnp.concatenate([dst, jnp.zeros((npad,), jnp.int32)])
    srcw = src_p.reshape(NW, CPW, CHUNK)
    dstw = dst_p.reshape(NW, CPW, CHUNK)
    zrows = jnp.zeros((RPS, D), jnp.float32)

    y1 = _tc_matmul(x, W1)
    (dp,) = _sc_deg(dstw, zrows)
    dcol = dp[:, :N, 0:1]
    (p,) = _sc_scatter(y1, srcw, dstw, zrows)
    y2, inv = _tc_mid(p, y1, dcol, b1.reshape(1, D), W2)
    (q,) = _sc_scatter(y2, srcw, dstw, zrows)
    return _tc_final(q, y2, inv, b2.reshape(1, D))
